# baseline (speedup 1.0000x reference)
import functools

import jax
import jax.numpy as jnp
from jax import lax
from jax.experimental import pallas as pl
from jax.experimental.pallas import tpu as pltpu
from jax.experimental.pallas import tpu_sc as plsc

B = 16384
N_SPARSE = 26
N_CONT = 13
VOCAB = 100000
EMB_DIM = 16
EMB_COLS = N_SPARSE * EMB_DIM
ROWS = B * N_SPARSE

NUM_CORES = 2
NUM_SUBCORES = 16
NW = NUM_CORES * NUM_SUBCORES
ROWS_PER_W = ROWS // NW
N_CHUNKS = 8
CHUNK = ROWS_PER_W // N_CHUNKS


def _sc_gather(table_flat, idx_flat):
    mesh = plsc.VectorSubcoreMesh(core_axis_name="c", subcore_axis_name="s")

    @functools.partial(
        pl.kernel,
        out_type=jax.ShapeDtypeStruct((ROWS, EMB_DIM), jnp.float32),
        mesh=mesh,
        scratch_types=[
            pltpu.VMEM((ROWS_PER_W,), jnp.int32),
            pltpu.VMEM((2, CHUNK, EMB_DIM), jnp.float32),
            pltpu.SemaphoreType.DMA,
            pltpu.SemaphoreType.DMA,
            pltpu.SemaphoreType.DMA,
        ],
        compiler_params=pltpu.CompilerParams(use_tc_tiling_on_sc=False),
    )
    def gather_kernel(table_hbm, idx_hbm, out_hbm, idx_v, rows_v, gsem0, gsem1, wsem):
        wid = lax.axis_index("s") * NUM_CORES + lax.axis_index("c")
        base = wid * ROWS_PER_W
        pltpu.sync_copy(idx_hbm.at[pl.ds(base, ROWS_PER_W)], idx_v)

        gsems = (gsem0, gsem1)
        g0 = pltpu.async_copy(
            table_hbm.at[idx_v.at[pl.ds(0, CHUNK)]], rows_v.at[0], gsems[0]
        )
        gathers = [g0, None]
        writes = [None, None]
        for ch in range(N_CHUNKS):
            buf = ch % 2
            nbuf = (ch + 1) % 2
            if ch + 1 < N_CHUNKS:
                if writes[nbuf] is not None:
                    writes[nbuf].wait()
                gathers[nbuf] = pltpu.async_copy(
                    table_hbm.at[idx_v.at[pl.ds((ch + 1) * CHUNK, CHUNK)]],
                    rows_v.at[nbuf],
                    gsems[nbuf],
                )
            gathers[buf].wait()
            writes[buf] = pltpu.async_copy(
                rows_v.at[buf], out_hbm.at[pl.ds(base + ch * CHUNK, CHUNK)], wsem
            )
        writes[0].wait()
        writes[1].wait()

    return gather_kernel(table_flat, idx_flat)


def _dense_body(
    emb_ref, cont_ref, w1a_ref, w1b_ref, b1_ref, w2_ref, b2_ref, w3_ref,
    fmwa_ref, fmwb_ref, fmva_ref, fmvb_ref, fmva2_ref, fmvb2_ref, bias_ref,
    o_ref,
):
    emb = emb_ref[...]
    cont = cont_ref[...]

    h1 = jnp.maximum(
        jnp.dot(emb, w1a_ref[...], preferred_element_type=jnp.float32)
        + jnp.dot(cont, w1b_ref[...], preferred_element_type=jnp.float32)
        + b1_ref[...],
        0.0,
    )
    h2 = jnp.maximum(
        jnp.dot(h1, w2_ref[...], preferred_element_type=jnp.float32) + b2_ref[...],
        0.0,
    )
    deep = jnp.sum(h2 * w3_ref[...], axis=1, keepdims=True)

    lin = (
        jnp.sum(emb * fmwa_ref[...], axis=1, keepdims=True)
        + jnp.sum(cont * fmwb_ref[...], axis=1, keepdims=True)
    )

    xv = jnp.dot(emb, fmva_ref[...], preferred_element_type=jnp.float32) + jnp.dot(
        cont, fmvb_ref[...], preferred_element_type=jnp.float32
    )
    x2v2 = jnp.dot(
        emb * emb, fmva2_ref[...], preferred_element_type=jnp.float32
    ) + jnp.dot(cont * cont, fmvb2_ref[...], preferred_element_type=jnp.float32)
    inter = 0.5 * jnp.sum(xv * xv - x2v2, axis=1, keepdims=True)

    o_ref[...] = lin + inter + deep + bias_ref[...]


def kernel(deep_sparse, deep_cont, emb_tables, fm_w, fm_b, fm_v, W1, b1, W2, b2, W3, b3):
    field_off = (jnp.arange(N_SPARSE, dtype=jnp.int32) * VOCAB)[None, :]
    idx_flat = (deep_sparse.astype(jnp.int32) + field_off).reshape(ROWS)
    table_flat = emb_tables.reshape(N_SPARSE * VOCAB, EMB_DIM)

    emb_rows = _sc_gather(table_flat, idx_flat)
    emb_x = emb_rows.reshape(B, EMB_COLS)

    cont = deep_cont.astype(jnp.float32)
    cont_p = jnp.pad(cont, ((0, 0), (0, 16 - N_CONT)))

    w1a = W1[:EMB_COLS]
    w1b = jnp.pad(W1[EMB_COLS:], ((0, 16 - N_CONT), (0, 0)))
    fmwa = fm_w[:EMB_COLS, 0][None, :]
    fmwb = jnp.pad(fm_w[EMB_COLS:, 0], (0, 16 - N_CONT))[None, :]
    fmva = fm_v[:EMB_COLS]
    fmvb = jnp.pad(fm_v[EMB_COLS:], ((0, 16 - N_CONT), (0, 0)))
    fmva2 = fmva * fmva
    fmvb2 = fmvb * fmvb
    w3row = W3[:, 0][None, :]
    b1r = b1[None, :]
    b2r = b2[None, :]
    bias = (fm_b + b3).reshape(1, 1)

    BM = 1024
    full = lambda shape: pl.BlockSpec(shape, lambda i: (0, 0))
    out = pl.pallas_call(
        _dense_body,
        grid=(B // BM,),
        in_specs=[
            pl.BlockSpec((BM, EMB_COLS), lambda i: (i, 0)),
            pl.BlockSpec((BM, 16), lambda i: (i, 0)),
            full((EMB_COLS, 256)),
            full((16, 256)),
            full((1, 256)),
            full((256, 128)),
            full((1, 128)),
            full((1, 128)),
            full((1, EMB_COLS)),
            full((1, 16)),
            full((EMB_COLS, 16)),
            full((16, 16)),
            full((EMB_COLS, 16)),
            full((16, 16)),
            full((1, 1)),
        ],
        out_specs=pl.BlockSpec((BM, 1), lambda i: (i, 0)),
        out_shape=jax.ShapeDtypeStruct((B, 1), jnp.float32),
    )(
        emb_x, cont_p, w1a, w1b, b1r, W2, b2r, w3row,
        fmwa, fmwb, fmva, fmvb, fmva2, fmvb2, bias,
    )
    return out


# speedup vs baseline: 4.2134x; 4.2134x over previous
import functools

import jax
import jax.numpy as jnp
from jax import lax
from jax.experimental import pallas as pl
from jax.experimental.pallas import tpu as pltpu
from jax.experimental.pallas import tpu_sc as plsc

B = 16384
N_SPARSE = 26
N_CONT = 13
VOCAB = 100000
EMB_DIM = 16
EMB_COLS = N_SPARSE * EMB_DIM

NUM_CORES = 2
NUM_SUBCORES = 16
NW = NUM_CORES * NUM_SUBCORES
COLS_PER_W = EMB_COLS // NW
CH = 2048
NCH = B // CH


def _sc_col_gather(tabT, idxT):
    mesh = plsc.VectorSubcoreMesh(core_axis_name="c", subcore_axis_name="s")

    @functools.partial(
        pl.kernel,
        out_type=jax.ShapeDtypeStruct((EMB_COLS, B), jnp.float32),
        mesh=mesh,
        scratch_types=[
            pltpu.VMEM((VOCAB,), jnp.float32),
            pltpu.VMEM((CH,), jnp.int32),
            pltpu.VMEM((CH,), jnp.float32),
            pltpu.SemaphoreType.DMA,
        ],
        compiler_params=pltpu.CompilerParams(
            use_tc_tiling_on_sc=True, needs_layout_passes=False
        ),
    )
    def col_gather(tabT_hbm, idxT_hbm, outT_hbm, vocab_v, idx_v, out_v, sem):
        wid = lax.axis_index("s") * NUM_CORES + lax.axis_index("c")
        for j in range(COLS_PER_W):
            r = wid * COLS_PER_W + j
            f = r // EMB_DIM
            pltpu.sync_copy(tabT_hbm.at[r], vocab_v)
            for c in range(NCH):
                pltpu.sync_copy(idxT_hbm.at[f, pl.ds(c * CH, CH)], idx_v)

                def body(i, _):
                    iv = idx_v[pl.ds(i * 16, 16)]
                    out_v[pl.ds(i * 16, 16)] = plsc.load_gather(vocab_v, [iv])
                    return 0

                lax.fori_loop(0, CH // 16, body, 0)
                pltpu.sync_copy(out_v, outT_hbm.at[r, pl.ds(c * CH, CH)])

    return col_gather(tabT, idxT)


def _dense_body(
    xt_ref, ct_ref, w1a_ref, w1b_ref, b1_ref, w2_ref, b2_ref, w3_ref,
    fmwa_ref, fmwb_ref, fmva_ref, fmvb_ref, bias_ref, o_ref,
):
    embT = xt_ref[...]
    contT = ct_ref[...]
    bf = jnp.bfloat16
    embTb = embT.astype(bf)
    contTb = contT.astype(bf)

    def dott(w, x):
        return lax.dot_general(
            w.astype(bf), x, (((0,), (0,)), ((), ())),
            preferred_element_type=jnp.float32,
        )

    h1T = jnp.maximum(
        dott(w1a_ref[...], embTb) + dott(w1b_ref[...], contTb) + b1_ref[...].T, 0.0
    )
    h2T = jnp.maximum(dott(w2_ref[...], h1T.astype(bf)) + b2_ref[...].T, 0.0)
    deepT = jnp.sum(h2T * w3_ref[...], axis=0, keepdims=True)

    linT = (
        jnp.sum(embT * fmwa_ref[...], axis=0, keepdims=True)
        + jnp.sum(contT * fmwb_ref[...], axis=0, keepdims=True)
    )

    fmva = fmva_ref[...]
    fmvb = fmvb_ref[...]
    xvT = dott(fmva, embTb) + dott(fmvb, contTb)
    x2v2T = dott(fmva * fmva, (embT * embT).astype(bf)) + dott(
        fmvb * fmvb, (contT * contT).astype(bf)
    )
    interT = 0.5 * jnp.sum(xvT * xvT - x2v2T, axis=0, keepdims=True)

    o_ref[...] = linT + interT + deepT + bias_ref[...]


def kernel(deep_sparse, deep_cont, emb_tables, fm_w, fm_b, fm_v, W1, b1, W2, b2, W3, b3):
    tabT = jnp.transpose(emb_tables, (0, 2, 1)).reshape(EMB_COLS, VOCAB)
    idxT = deep_sparse.T.astype(jnp.int32)

    xT = _sc_col_gather(tabT, idxT)

    contT = jnp.pad(deep_cont.astype(jnp.float32).T, ((0, 16 - N_CONT), (0, 0)))

    w1a = W1[:EMB_COLS]
    w1b = jnp.pad(W1[EMB_COLS:], ((0, 16 - N_CONT), (0, 0)))
    fmwa = fm_w[:EMB_COLS]
    fmwb = jnp.pad(fm_w[EMB_COLS:], ((0, 16 - N_CONT), (0, 0)))
    fmva = fm_v[:EMB_COLS]
    fmvb = jnp.pad(fm_v[EMB_COLS:], ((0, 16 - N_CONT), (0, 0)))
    b1r = b1[None, :]
    b2r = b2[None, :]
    bias = (fm_b + b3).reshape(1, 1)

    BN = 2048
    full = lambda shape: pl.BlockSpec(shape, lambda i: (0, 0))
    outT = pl.pallas_call(
        _dense_body,
        grid=(B // BN,),
        in_specs=[
            pl.BlockSpec((EMB_COLS, BN), lambda i: (0, i)),
            pl.BlockSpec((16, BN), lambda i: (0, i)),
            full((EMB_COLS, 256)),
            full((16, 256)),
            full((1, 256)),
            full((256, 128)),
            full((1, 128)),
            full((128, 1)),
            full((EMB_COLS, 1)),
            full((16, 1)),
            full((EMB_COLS, 16)),
            full((16, 16)),
            full((1, 1)),
        ],
        out_specs=pl.BlockSpec((1, BN), lambda i: (0, i)),
        out_shape=jax.ShapeDtypeStruct((1, B), jnp.float32),
    )(
        xT, contT, w1a, w1b, b1r, W2, b2r, W3,
        fmwa, fmwb, fmva, fmvb, bias,
    )
    return outT.reshape(B, 1)


# speedup vs baseline: 4.8755x; 1.1571x over previous
import functools

import jax
import jax.numpy as jnp
from jax import lax
from jax.experimental import pallas as pl
from jax.experimental.pallas import tpu as pltpu
from jax.experimental.pallas import tpu_sc as plsc

B = 16384
N_SPARSE = 26
N_CONT = 13
VOCAB = 100000
EMB_DIM = 16
EMB_COLS = N_SPARSE * EMB_DIM

NUM_CORES = 2
NUM_SUBCORES = 16
NW = NUM_CORES * NUM_SUBCORES
COLS_PER_W = EMB_COLS // NW
CH = 2048
NCH = B // CH


def _sc_col_gather(tabT, idxT):
    mesh = plsc.VectorSubcoreMesh(core_axis_name="c", subcore_axis_name="s")

    @functools.partial(
        pl.kernel,
        out_type=jax.ShapeDtypeStruct((EMB_COLS, B), jnp.float32),
        mesh=mesh,
        scratch_types=[
            pltpu.VMEM((VOCAB,), jnp.float32),
            pltpu.VMEM((B,), jnp.int32),
            pltpu.VMEM((2, CH), jnp.float32),
            pltpu.SemaphoreType.DMA,
        ],
        compiler_params=pltpu.CompilerParams(
            use_tc_tiling_on_sc=True, needs_layout_passes=False
        ),
    )
    def col_gather(tabT_hbm, idxT_hbm, outT_hbm, vocab_v, idxf_v, out_v, wsem):
        wid = lax.axis_index("s") * NUM_CORES + lax.axis_index("c")
        UNROLL = 8
        writes = [None, None]
        for j in range(COLS_PER_W):
            r = wid * COLS_PER_W + j
            f = r // EMB_DIM
            if j == 0:
                pltpu.sync_copy(idxT_hbm.at[f], idxf_v)
            else:
                f_prev = (wid * COLS_PER_W + j - 1) // EMB_DIM

                @pl.when(f != f_prev)
                def _():
                    pltpu.sync_copy(idxT_hbm.at[f], idxf_v)

            pltpu.sync_copy(tabT_hbm.at[r], vocab_v)
            for c in range(NCH):
                buf = c % 2
                if writes[buf] is not None:
                    writes[buf].wait()
                    writes[buf] = None

                def body(i, _, buf=buf, c=c):
                    for u in range(UNROLL):
                        off = i * (16 * UNROLL) + u * 16
                        iv = idxf_v[pl.ds(c * CH + off, 16)]
                        out_v[buf, pl.ds(off, 16)] = plsc.load_gather(
                            vocab_v, [iv]
                        )
                    return 0

                lax.fori_loop(0, CH // (16 * UNROLL), body, 0)
                writes[buf] = pltpu.async_copy(
                    out_v.at[buf], outT_hbm.at[r, pl.ds(c * CH, CH)], wsem
                )
        for w in writes:
            if w is not None:
                w.wait()

    return col_gather(tabT, idxT)


def _dense_body(
    xt_ref, ct_ref, w1a_ref, w1b_ref, b1_ref, w2_ref, b2_ref, w3_ref,
    fmwa_ref, fmwb_ref, fmva_ref, fmvb_ref, bias_ref, o_ref,
):
    embT = xt_ref[...]
    contT = ct_ref[...]
    bf = jnp.bfloat16
    embTb = embT.astype(bf)
    contTb = contT.astype(bf)

    def dott(w, x):
        return lax.dot_general(
            w.astype(bf), x, (((0,), (0,)), ((), ())),
            preferred_element_type=jnp.float32,
        )

    h1T = jnp.maximum(
        dott(w1a_ref[...], embTb) + dott(w1b_ref[...], contTb) + b1_ref[...].T, 0.0
    )
    h2T = jnp.maximum(dott(w2_ref[...], h1T.astype(bf)) + b2_ref[...].T, 0.0)
    deepT = jnp.sum(h2T * w3_ref[...], axis=0, keepdims=True)

    linT = (
        jnp.sum(embT * fmwa_ref[...], axis=0, keepdims=True)
        + jnp.sum(contT * fmwb_ref[...], axis=0, keepdims=True)
    )

    fmva = fmva_ref[...]
    fmvb = fmvb_ref[...]
    xvT = dott(fmva, embTb) + dott(fmvb, contTb)
    x2v2T = dott(fmva * fmva, (embT * embT).astype(bf)) + dott(
        fmvb * fmvb, (contT * contT).astype(bf)
    )
    interT = 0.5 * jnp.sum(xvT * xvT - x2v2T, axis=0, keepdims=True)

    o_ref[...] = linT + interT + deepT + bias_ref[...]


def kernel(deep_sparse, deep_cont, emb_tables, fm_w, fm_b, fm_v, W1, b1, W2, b2, W3, b3):
    tabT = jnp.transpose(emb_tables, (0, 2, 1)).reshape(EMB_COLS, VOCAB)
    idxT = deep_sparse.T.astype(jnp.int32)

    xT = _sc_col_gather(tabT, idxT)

    contT = jnp.pad(deep_cont.astype(jnp.float32).T, ((0, 16 - N_CONT), (0, 0)))

    w1a = W1[:EMB_COLS]
    w1b = jnp.pad(W1[EMB_COLS:], ((0, 16 - N_CONT), (0, 0)))
    fmwa = fm_w[:EMB_COLS]
    fmwb = jnp.pad(fm_w[EMB_COLS:], ((0, 16 - N_CONT), (0, 0)))
    fmva = fm_v[:EMB_COLS]
    fmvb = jnp.pad(fm_v[EMB_COLS:], ((0, 16 - N_CONT), (0, 0)))
    b1r = b1[None, :]
    b2r = b2[None, :]
    bias = (fm_b + b3).reshape(1, 1)

    BN = 2048
    full = lambda shape: pl.BlockSpec(shape, lambda i: (0, 0))
    outT = pl.pallas_call(
        _dense_body,
        grid=(B // BN,),
        in_specs=[
            pl.BlockSpec((EMB_COLS, BN), lambda i: (0, i)),
            pl.BlockSpec((16, BN), lambda i: (0, i)),
            full((EMB_COLS, 256)),
            full((16, 256)),
            full((1, 256)),
            full((256, 128)),
            full((1, 128)),
            full((128, 1)),
            full((EMB_COLS, 1)),
            full((16, 1)),
            full((EMB_COLS, 16)),
            full((16, 16)),
            full((1, 1)),
        ],
        out_specs=pl.BlockSpec((1, BN), lambda i: (0, i)),
        out_shape=jax.ShapeDtypeStruct((1, B), jnp.float32),
    )(
        xT, contT, w1a, w1b, b1r, W2, b2r, W3,
        fmwa, fmwb, fmva, fmvb, bias,
    )
    return outT.reshape(B, 1)
